# baseline (speedup 1.0000x reference)
import jax
import jax.numpy as jnp
from jax.experimental import pallas as pl

FEAT = 64
SEQ = 1024
NS0 = 24
NS1 = 7
LPS1 = 24
BN = 32


def _tables_kernel(z_ref, w0_ref, b0_ref, w1_ref, b1_ref, p0_ref, p1_ref):
    z = z_ref[...]
    p0_ref[...] = (
        jnp.dot(z, w0_ref[...], preferred_element_type=jnp.float32) + b0_ref[...]
    )
    p1_ref[...] = (
        jnp.dot(z, w1_ref[...], preferred_element_type=jnp.float32) + b1_ref[...]
    )


def _expand_kernel(p0_ref, p1_ref, out_ref):
    t0 = jax.lax.broadcasted_iota(jnp.int32, (NS0, SEQ), 1)
    s0 = jax.lax.broadcasted_iota(jnp.int32, (NS0, SEQ), 0)
    g0 = ((t0 % NS0) == s0).astype(jnp.float32)
    t1 = jax.lax.broadcasted_iota(jnp.int32, (NS1, SEQ), 1)
    s1 = jax.lax.broadcasted_iota(jnp.int32, (NS1, SEQ), 0)
    g1 = (((t1 // LPS1) % NS1) == s1).astype(jnp.float32)
    out_ref[...] = jnp.dot(
        p0_ref[...], g0, preferred_element_type=jnp.float32
    ) + jnp.dot(p1_ref[...], g1, preferred_element_type=jnp.float32)


def kernel(z, W0, b0, W1, b1):
    N, LATENT = z.shape
    b0r = b0.reshape(1, FEAT * NS0)
    b1r = b1.reshape(1, FEAT * NS1)
    p0, p1 = pl.pallas_call(
        _tables_kernel,
        grid=(1,),
        in_specs=[
            pl.BlockSpec((N, LATENT), lambda i: (0, 0)),
            pl.BlockSpec((LATENT, FEAT * NS0), lambda i: (0, 0)),
            pl.BlockSpec((1, FEAT * NS0), lambda i: (0, 0)),
            pl.BlockSpec((LATENT, FEAT * NS1), lambda i: (0, 0)),
            pl.BlockSpec((1, FEAT * NS1), lambda i: (0, 0)),
        ],
        out_specs=[
            pl.BlockSpec((N, FEAT * NS0), lambda i: (0, 0)),
            pl.BlockSpec((N, FEAT * NS1), lambda i: (0, 0)),
        ],
        out_shape=[
            jax.ShapeDtypeStruct((N, FEAT * NS0), jnp.float32),
            jax.ShapeDtypeStruct((N, FEAT * NS1), jnp.float32),
        ],
    )(z, W0, b0r, W1, b1r)
    p0r = p0.reshape(N * FEAT, NS0)
    p1r = p1.reshape(N * FEAT, NS1)
    vals = pl.pallas_call(
        _expand_kernel,
        grid=(N // BN,),
        in_specs=[
            pl.BlockSpec((BN * FEAT, NS0), lambda i: (i, 0)),
            pl.BlockSpec((BN * FEAT, NS1), lambda i: (i, 0)),
        ],
        out_specs=pl.BlockSpec((BN * FEAT, SEQ), lambda i: (i, 0)),
        out_shape=jax.ShapeDtypeStruct((N * FEAT, SEQ), jnp.float32),
    )(p0r, p1r)
    return jnp.transpose(vals.reshape(N, FEAT, SEQ), (0, 2, 1))


# speedup vs baseline: 1.0113x; 1.0113x over previous
import jax
import jax.numpy as jnp
from jax.experimental import pallas as pl

FEAT = 64
SEQ = 1024
NS0 = 24
NS1 = 7
LPS1 = 24
BN = 64


def _tables_kernel(z_ref, w0_ref, b0_ref, w1_ref, b1_ref, p0_ref, p1_ref):
    z = z_ref[...]
    p0_ref[...] = (
        jnp.dot(z, w0_ref[...], preferred_element_type=jnp.float32) + b0_ref[...]
    )
    p1_ref[...] = (
        jnp.dot(z, w1_ref[...], preferred_element_type=jnp.float32) + b1_ref[...]
    )


def _expand_kernel(p0_ref, p1_ref, out_ref):
    t0 = jax.lax.broadcasted_iota(jnp.int32, (NS0, SEQ), 1)
    s0 = jax.lax.broadcasted_iota(jnp.int32, (NS0, SEQ), 0)
    g0 = ((t0 % NS0) == s0).astype(jnp.float32)
    t1 = jax.lax.broadcasted_iota(jnp.int32, (NS1, SEQ), 1)
    s1 = jax.lax.broadcasted_iota(jnp.int32, (NS1, SEQ), 0)
    g1 = (((t1 // LPS1) % NS1) == s1).astype(jnp.float32)
    out_ref[...] = jnp.dot(
        p0_ref[...], g0, preferred_element_type=jnp.float32
    ) + jnp.dot(p1_ref[...], g1, preferred_element_type=jnp.float32)


def kernel(z, W0, b0, W1, b1):
    N, LATENT = z.shape
    b0r = b0.reshape(1, FEAT * NS0)
    b1r = b1.reshape(1, FEAT * NS1)
    p0, p1 = pl.pallas_call(
        _tables_kernel,
        grid=(1,),
        in_specs=[
            pl.BlockSpec((N, LATENT), lambda i: (0, 0)),
            pl.BlockSpec((LATENT, FEAT * NS0), lambda i: (0, 0)),
            pl.BlockSpec((1, FEAT * NS0), lambda i: (0, 0)),
            pl.BlockSpec((LATENT, FEAT * NS1), lambda i: (0, 0)),
            pl.BlockSpec((1, FEAT * NS1), lambda i: (0, 0)),
        ],
        out_specs=[
            pl.BlockSpec((N, FEAT * NS0), lambda i: (0, 0)),
            pl.BlockSpec((N, FEAT * NS1), lambda i: (0, 0)),
        ],
        out_shape=[
            jax.ShapeDtypeStruct((N, FEAT * NS0), jnp.float32),
            jax.ShapeDtypeStruct((N, FEAT * NS1), jnp.float32),
        ],
    )(z, W0, b0r, W1, b1r)
    p0r = p0.reshape(N * FEAT, NS0)
    p1r = p1.reshape(N * FEAT, NS1)
    vals = pl.pallas_call(
        _expand_kernel,
        grid=(N // BN,),
        in_specs=[
            pl.BlockSpec((BN * FEAT, NS0), lambda i: (i, 0)),
            pl.BlockSpec((BN * FEAT, NS1), lambda i: (i, 0)),
        ],
        out_specs=pl.BlockSpec((BN * FEAT, SEQ), lambda i: (i, 0)),
        out_shape=jax.ShapeDtypeStruct((N * FEAT, SEQ), jnp.float32),
    )(p0r, p1r)
    return jnp.transpose(vals.reshape(N, FEAT, SEQ), (0, 2, 1))
